# Initial kernel scaffold; baseline (speedup 1.0000x reference)
import jax
import jax.numpy as jnp
from jax.experimental import pallas as pl


def kernel(x, t, loc_table_0, loc_table_1, time_table_0, time_table_1):
    raise NotImplementedError("write your pallas kernel here")


# speedup vs baseline: 3.4224x; 3.4224x over previous
import functools

import jax
import jax.numpy as jnp
from jax import lax
from jax.experimental import pallas as pl
from jax.experimental.pallas import tpu as pltpu
from jax.experimental.pallas import tpu_sc as plsc

B = 4096
L = 200
N = B * L
D_LOC = 64
D_TIME = 16
D_OUT = D_LOC + D_TIME
T0_ROWS = 1001
T1_ROWS = 101

NC = 2
NS = 16
NW = NC * NS
ROWS_PER_W = N // NW
CHUNK = 256
NCHUNKS = ROWS_PER_W // CHUNK
GATHER = 128


def _make_sc_kernel():
    mesh = plsc.VectorSubcoreMesh(core_axis_name="c", subcore_axis_name="s")

    @functools.partial(
        pl.kernel,
        mesh=mesh,
        out_type=jax.ShapeDtypeStruct((N * D_OUT,), jnp.float32),
        compiler_params=pltpu.CompilerParams(
            needs_layout_passes=False, use_tc_tiling_on_sc=False),
        scratch_types=[
            pltpu.VMEM((CHUNK,), jnp.int32),
            pltpu.VMEM((CHUNK,), jnp.int32),
            pltpu.VMEM((CHUNK,), jnp.int32),
            pltpu.VMEM((CHUNK,), jnp.int32),
            pltpu.VMEM((CHUNK, D_LOC), jnp.float32),
            pltpu.VMEM((CHUNK, D_LOC), jnp.float32),
            pltpu.VMEM((T0_ROWS * D_TIME,), jnp.float32),
            pltpu.VMEM((T1_ROWS * D_TIME,), jnp.float32),
            pltpu.VMEM((CHUNK * D_OUT,), jnp.float32),
            pltpu.SemaphoreType.DMA,
        ],
    )
    def k(x0h, x1h, t0h, t1h, loc0h, loc1h, tt0h, tt1h, outh,
          x0v, x1v, t0v, t1v, r0, r1, tt0v, tt1v, ob, sem):
        wid = lax.axis_index("s") * NC + lax.axis_index("c")
        base0 = wid * ROWS_PER_W

        pltpu.sync_copy(tt0h, tt0v)
        pltpu.sync_copy(tt1h, tt1v)

        lane = lax.iota(jnp.int32, 16)

        def chunk_body(ci, carry):
            base = base0 + ci * CHUNK
            pltpu.sync_copy(x0h.at[pl.ds(base, CHUNK)], x0v)
            pltpu.sync_copy(x1h.at[pl.ds(base, CHUNK)], x1v)
            pltpu.sync_copy(t0h.at[pl.ds(base, CHUNK)], t0v)
            pltpu.sync_copy(t1h.at[pl.ds(base, CHUNK)], t1v)

            cps = []
            for g in range(CHUNK // GATHER):
                s = pl.ds(g * GATHER, GATHER)
                cps.append(pltpu.async_copy(loc0h.at[x0v.at[s]], r0.at[s], sem))
                cps.append(pltpu.async_copy(loc1h.at[x1v.at[s]], r1.at[s], sem))
            for cp in cps:
                cp.wait()

            def loc_body(i, c):
                for j in range(D_LOC // 16):
                    s = pl.ds(j * 16, 16)
                    ob[pl.ds(i * D_OUT + j * 16, 16)] = r0[i, s] + r1[i, s]
                return c
            lax.fori_loop(0, CHUNK, loc_body, 0, unroll=2)

            def time_body(g, c):
                tv0 = t0v[pl.ds(g * 16, 16)]
                tv1 = t1v[pl.ds(g * 16, 16)]
                f0 = tv0 * D_TIME
                f1 = tv1 * D_TIME
                pos = (lane + g * 16) * D_OUT + D_LOC
                for j in range(D_TIME):
                    v0 = plsc.load_gather(tt0v, [f0 + j])
                    v1 = plsc.load_gather(tt1v, [f1 + j])
                    plsc.store_scatter(ob, [pos + j], v0 + v1)
                return c
            lax.fori_loop(0, CHUNK // 16, time_body, 0)

            pltpu.sync_copy(ob, outh.at[pl.ds(base * D_OUT, CHUNK * D_OUT)])
            return carry

        lax.fori_loop(0, NCHUNKS, chunk_body, 0)

    return k


_sc_lookup = _make_sc_kernel()


def kernel(x, t, loc_table_0, loc_table_1, time_table_0, time_table_1):
    x = x.astype(jnp.int32)
    t = t.astype(jnp.int32)
    x0 = x[..., 0].reshape(N)
    x1 = x[..., 1].reshape(N)
    t0 = t[..., 0].reshape(N)
    t1 = t[..., 1].reshape(N)
    out = _sc_lookup(x0, x1, t0, t1, loc_table_0, loc_table_1,
                     time_table_0.reshape(-1), time_table_1.reshape(-1))
    return out.reshape(B, L, D_OUT)
